# Initial kernel scaffold; baseline (speedup 1.0000x reference)
import jax
import jax.numpy as jnp
from jax.experimental import pallas as pl


def kernel(x, edge_index, batch, edge_weight, W1, b1, W2, b2, W3, b3):
    raise NotImplementedError("write your pallas kernel here")


# speedup vs baseline: 6.7100x; 6.7100x over previous
import functools

import jax
import jax.numpy as jnp
from jax import lax
from jax.experimental import pallas as pl
from jax.experimental.pallas import tpu as pltpu
from jax.experimental.pallas import tpu_sc as plsc

N = 10000
E = 320000
D = 128
G = 64

NC = 2
NS = 16
NW = NC * NS

NPAD = 10240
NSLC = NPAD // NS
EPW = E // NW
EPT = E // NS
KE = 80
ROWS_PT = NPAD // NS

_mesh = plsc.VectorSubcoreMesh(core_axis_name="c", subcore_axis_name="s")
_sc_params = pltpu.CompilerParams(needs_layout_passes=False)


def _rsqrt16(d):
    i = lax.bitcast_convert_type(d, jnp.int32)
    i = jnp.int32(0x5F3759DF) - lax.shift_right_logical(i, 1)
    y = lax.bitcast_convert_type(i, jnp.float32)
    half = d * 0.5
    for _ in range(4):
        y = y * (1.5 - half * y * y)
    return y


def _sc_prep_body(row_hbm, col_hbm, ew_hbm, norm_hbm, selfw_hbm,
                  row_v, col_v, ew_v, slc_v, slc2_v, dinv_v, nbuf_v,
                  deg_sh, dinv_sh, sem):
    cid = lax.axis_index("c")
    sid = lax.axis_index("s")
    wid = sid * NC + cid

    def _fill_one(i, _):
        slc_v[pl.ds(i * 16, 16)] = jnp.full((16,), 1.0, jnp.float32)
        return 0
    lax.fori_loop(0, NSLC // 16, _fill_one, 0, unroll=8)
    pltpu.sync_copy(slc_v, deg_sh.at[pl.ds(sid * NSLC, NSLC)])
    plsc.subcore_barrier()

    def _deg_chunk(ci, _):
        base = pl.multiple_of(sid * EPT + ci * KE, 8)
        pltpu.sync_copy(col_hbm.at[pl.ds(base, KE)], col_v)
        pltpu.sync_copy(ew_hbm.at[pl.ds(base, KE)], ew_v)
        pltpu.sync_copy(ew_v, deg_sh.at[col_v], add=True)
        return 0
    lax.fori_loop(0, EPT // KE, _deg_chunk, 0)
    plsc.subcore_barrier()

    pltpu.sync_copy(deg_sh.at[pl.ds(sid * NSLC, NSLC)], slc_v)
    def _rsq(i, _):
        d = slc_v[pl.ds(i * 16, 16)]
        y = _rsqrt16(d)
        slc_v[pl.ds(i * 16, 16)] = y
        slc2_v[pl.ds(i * 16, 16)] = y * y
        return 0
    lax.fori_loop(0, NSLC // 16, _rsq, 0, unroll=4)
    pltpu.sync_copy(slc_v, dinv_sh.at[pl.ds(sid * NSLC, NSLC)])

    @pl.when(cid == 0)
    def _():
        pltpu.sync_copy(slc2_v, selfw_hbm.at[pl.ds(sid * NSLC, NSLC)])

    plsc.subcore_barrier()
    pltpu.sync_copy(dinv_sh, dinv_v)

    def _norm_chunk(ci, _):
        base = pl.multiple_of(wid * EPW + ci * KE, 8)
        pltpu.sync_copy(col_hbm.at[pl.ds(base, KE)], col_v)
        pltpu.sync_copy(row_hbm.at[pl.ds(base, KE)], row_v)
        pltpu.sync_copy(ew_hbm.at[pl.ds(base, KE)], ew_v)
        def _inner(i, _):
            r16 = row_v[pl.ds(i * 16, 16)]
            c16 = col_v[pl.ds(i * 16, 16)]
            w16 = ew_v[pl.ds(i * 16, 16)]
            dr = plsc.load_gather(dinv_v, [r16])
            dc = plsc.load_gather(dinv_v, [c16])
            nbuf_v[pl.ds(i * 16, 16)] = dr * w16 * dc
            return 0
        lax.fori_loop(0, KE // 16, _inner, 0, unroll=5)
        pltpu.sync_copy(nbuf_v, norm_hbm.at[pl.ds(base, KE)])
        return 0
    lax.fori_loop(0, EPW // KE, _norm_chunk, 0)


@jax.jit
def _sc_prep(row, col, ew):
    return pl.kernel(
        _sc_prep_body,
        out_type=(
            jax.ShapeDtypeStruct((E,), jnp.float32),
            jax.ShapeDtypeStruct((NPAD,), jnp.float32),
        ),
        mesh=_mesh,
        scratch_types=[
            pltpu.VMEM((KE,), jnp.int32),
            pltpu.VMEM((KE,), jnp.int32),
            pltpu.VMEM((KE,), jnp.float32),
            pltpu.VMEM((NSLC,), jnp.float32),
            pltpu.VMEM((NSLC,), jnp.float32),
            pltpu.VMEM((NPAD,), jnp.float32),
            pltpu.VMEM((KE,), jnp.float32),
            pltpu.VMEM_SHARED((NPAD,), jnp.float32),
            pltpu.VMEM_SHARED((NPAD,), jnp.float32),
            pltpu.SemaphoreType.DMA,
        ],
        compiler_params=_sc_params,
    )(row, col, ew)


def _sc_mp_body(xw_hbm, row_hbm, col_hbm, norm_hbm, parts_hbm,
                row_v, col_v, nv_v, gbuf_v, acc_sh, sem):
    cid = lax.axis_index("c")
    sid = lax.axis_index("s")
    wid = sid * NC + cid

    def _z(i, _):
        for j in range(8):
            gbuf_v[i, pl.ds(j * 16, 16)] = jnp.zeros((16,), jnp.float32)
        return 0
    lax.fori_loop(0, KE, _z, 0, unroll=8)
    r0 = sid * ROWS_PT
    for k in range(ROWS_PT // KE):
        pltpu.sync_copy(gbuf_v, acc_sh.at[pl.ds(r0 + k * KE, KE)])
    plsc.subcore_barrier()

    def _chunk(ci, _):
        base = pl.multiple_of(wid * EPW + ci * KE, 8)
        pltpu.sync_copy(row_hbm.at[pl.ds(base, KE)], row_v)
        pltpu.sync_copy(col_hbm.at[pl.ds(base, KE)], col_v)
        pltpu.sync_copy(norm_hbm.at[pl.ds(base, KE)], nv_v)
        pltpu.async_copy(xw_hbm.at[row_v], gbuf_v, sem).wait()

        def _scale(e, _):
            w = plsc.load_gather(nv_v, [jnp.full((16,), e, jnp.int32)])
            g = gbuf_v.at[e]
            for j in range(8):
                g[pl.ds(j * 16, 16)] = g[pl.ds(j * 16, 16)] * w
            return 0
        lax.fori_loop(0, KE, _scale, 0, unroll=4)

        pltpu.sync_copy(gbuf_v, acc_sh.at[col_v], add=True)
        return 0
    lax.fori_loop(0, EPW // KE, _chunk, 0)
    plsc.subcore_barrier()

    pltpu.sync_copy(acc_sh.at[pl.ds(r0, ROWS_PT)],
                    parts_hbm.at[cid, pl.ds(r0, ROWS_PT)])


@jax.jit
def _sc_mp(xw, row, col, norm):
    return pl.kernel(
        _sc_mp_body,
        out_type=jax.ShapeDtypeStruct((NC, NPAD, D), jnp.float32),
        mesh=_mesh,
        scratch_types=[
            pltpu.VMEM((KE,), jnp.int32),
            pltpu.VMEM((KE,), jnp.int32),
            pltpu.VMEM((KE,), jnp.float32),
            pltpu.VMEM((KE, D), jnp.float32),
            pltpu.VMEM_SHARED((NPAD, D), jnp.float32),
            pltpu.SemaphoreType.DMA,
        ],
        compiler_params=_sc_params,
    )(xw, row, col, norm)


RB = 1000
NRB = N // RB


def _mm_first_body(x_ref, w_ref, o_ref):
    o_ref[...] = jnp.dot(x_ref[...], w_ref[...],
                         preferred_element_type=jnp.float32,
                         precision=lax.Precision.HIGHEST)


@jax.jit
def _tc_mm_first(x, W):
    return pl.pallas_call(
        _mm_first_body,
        grid=(NRB,),
        in_specs=[
            pl.BlockSpec((RB, D), lambda i: (i, 0)),
            pl.BlockSpec((D, D), lambda i: (0, 0)),
        ],
        out_specs=pl.BlockSpec((RB, D), lambda i: (i, 0)),
        out_shape=jax.ShapeDtypeStruct((N, D), jnp.float32),
    )(x, W)


def _mm_mid_body(p_ref, xw_ref, sw_ref, b_ref, w_ref, o_ref):
    h = p_ref[0] + p_ref[1] + sw_ref[...] * xw_ref[...] + b_ref[...]
    h = jnp.maximum(h, 0.0)
    o_ref[...] = jnp.dot(h, w_ref[...],
                         preferred_element_type=jnp.float32,
                         precision=lax.Precision.HIGHEST)


@jax.jit
def _tc_mm_mid(parts, xw, sw, b, W):
    return pl.pallas_call(
        _mm_mid_body,
        grid=(NRB,),
        in_specs=[
            pl.BlockSpec((NC, RB, D), lambda i: (0, i, 0)),
            pl.BlockSpec((RB, D), lambda i: (i, 0)),
            pl.BlockSpec((RB, 1), lambda i: (i, 0)),
            pl.BlockSpec((1, D), lambda i: (0, 0)),
            pl.BlockSpec((D, D), lambda i: (0, 0)),
        ],
        out_specs=pl.BlockSpec((RB, D), lambda i: (i, 0)),
        out_shape=jax.ShapeDtypeStruct((N, D), jnp.float32),
    )(parts, xw, sw, b, W)


def _pool_body(p_ref, xw_ref, sw_ref, b_ref, bat_ref, o_ref, acc, cnt):
    i = pl.program_id(0)

    @pl.when(i == 0)
    def _():
        acc[...] = jnp.zeros_like(acc)
        cnt[...] = jnp.zeros_like(cnt)

    h = p_ref[0] + p_ref[1] + sw_ref[...] * xw_ref[...] + b_ref[...]
    gids = lax.broadcasted_iota(jnp.int32, (G, RB), 0)
    onehot = (gids == bat_ref[0]).astype(jnp.float32)
    acc[...] += jnp.dot(onehot, h, preferred_element_type=jnp.float32,
                        precision=lax.Precision.HIGHEST)
    cnt[...] += jnp.broadcast_to(jnp.sum(onehot, axis=1, keepdims=True),
                                 (G, D))

    @pl.when(i == NRB - 1)
    def _():
        o_ref[...] = acc[...] / jnp.maximum(cnt[...], 1.0)


@jax.jit
def _tc_pool(parts, xw, sw, b, batch2d):
    return pl.pallas_call(
        _pool_body,
        grid=(NRB,),
        in_specs=[
            pl.BlockSpec((NC, RB, D), lambda i: (0, i, 0)),
            pl.BlockSpec((RB, D), lambda i: (i, 0)),
            pl.BlockSpec((RB, 1), lambda i: (i, 0)),
            pl.BlockSpec((1, D), lambda i: (0, 0)),
            pl.BlockSpec((1, 1, RB), lambda i: (i, 0, 0)),
        ],
        out_specs=pl.BlockSpec((G, D), lambda i: (0, 0)),
        out_shape=jax.ShapeDtypeStruct((G, D), jnp.float32),
        scratch_shapes=[
            pltpu.VMEM((G, D), jnp.float32),
            pltpu.VMEM((G, D), jnp.float32),
        ],
    )(parts, xw, sw, b, batch2d)


def kernel(x, edge_index, batch, edge_weight, W1, b1, W2, b2, W3, b3):
    ei = edge_index.astype(jnp.int32)
    row = ei[0]
    col = ei[1]
    ew = edge_weight.astype(jnp.float32)

    norm, selfw = _sc_prep(row, col, ew)
    sw = selfw[:N].reshape(N, 1)
    b1r = b1.reshape(1, D)
    b2r = b2.reshape(1, D)
    b3r = b3.reshape(1, D)
    bat2d = batch.astype(jnp.int32).reshape(N // RB, 1, RB)

    xw1 = _tc_mm_first(x, W1)
    p1 = _sc_mp(xw1, row, col, norm)
    xw2 = _tc_mm_mid(p1, xw1, sw, b1r, W2)
    p2 = _sc_mp(xw2, row, col, norm)
    xw3 = _tc_mm_mid(p2, xw2, sw, b2r, W3)
    p3 = _sc_mp(xw3, row, col, norm)
    return _tc_pool(p3, xw3, sw, b3r, bat2d)
